# Initial kernel scaffold; baseline (speedup 1.0000x reference)
import jax
import jax.numpy as jnp
from jax.experimental import pallas as pl


def kernel(x, ei, W1, b1, W2, b2):
    raise NotImplementedError("write your pallas kernel here")


# speedup vs baseline: 11.1500x; 11.1500x over previous
import functools

import jax
import jax.numpy as jnp
from jax import lax
from jax.experimental import pallas as pl
from jax.experimental.pallas import tpu as pltpu
from jax.experimental.pallas import tpu_sc as plsc

NC = 2
NS = 16
NW = NC * NS
B = 128
L = 16

_mesh = plsc.VectorSubcoreMesh(core_axis_name="core", subcore_axis_name="subcore")


def _zero_rows(buf, nrows, width):
    @pl.loop(0, nrows)
    def _(i):
        for k in range(width // L):
            buf[i, pl.ds(k * L, L)] = jnp.zeros((L,), jnp.float32)


def _sc_degree(dst_r, NP, CH, F):
    @functools.partial(
        pl.kernel,
        out_type=jax.ShapeDtypeStruct((NC, NP, F), jnp.float32),
        mesh=_mesh,
        scratch_types=[
            pltpu.VMEM((CH, B), jnp.int32),
            pltpu.VMEM((B, F), jnp.float32),
            pltpu.VMEM_SHARED((NP, F), jnp.float32),
        ],
    )
    def deg_kernel(dst_hbm, out_hbm, idx_v, buf_v, acc_sh):
        cid = lax.axis_index("core")
        sid = lax.axis_index("subcore")
        wid = cid * NS + sid
        rows_per_tile = NP // NS
        chunks_per_tile = rows_per_tile // B

        pltpu.sync_copy(dst_hbm.at[wid], idx_v)

        _zero_rows(buf_v, B, F)
        @pl.loop(0, chunks_per_tile)
        def _(j):
            pltpu.sync_copy(buf_v, acc_sh.at[pl.ds(sid * rows_per_tile + j * B, B)])
        plsc.subcore_barrier()

        @pl.loop(0, B)
        def _(i):
            for k in range(F // L):
                buf_v[i, pl.ds(k * L, L)] = jnp.ones((L,), jnp.float32)

        @pl.loop(0, CH)
        def _(j):
            pltpu.sync_copy(buf_v, acc_sh.at[idx_v.at[j]], add=True)
        plsc.subcore_barrier()

        @pl.loop(0, chunks_per_tile)
        def _(j):
            base = sid * rows_per_tile + j * B
            pltpu.sync_copy(acc_sh.at[pl.ds(base, B)], buf_v)
            pltpu.sync_copy(buf_v, out_hbm.at[cid].at[pl.ds(base, B)])

    return deg_kernel(dst_r)


def _sc_aggregate(h, src_r, dst_r, NP, CH, F):
    @functools.partial(
        pl.kernel,
        out_type=jax.ShapeDtypeStruct((NC, NP, F), jnp.float32),
        mesh=_mesh,
        scratch_types=[
            pltpu.VMEM((CH, B), jnp.int32),
            pltpu.VMEM((CH, B), jnp.int32),
            pltpu.VMEM((B, F), jnp.float32),
            pltpu.VMEM_SHARED((NP, F), jnp.float32),
        ],
    )
    def agg_kernel(h_hbm, src_hbm, dst_hbm, out_hbm, sidx, didx, g0, acc_sh):
        cid = lax.axis_index("core")
        sid = lax.axis_index("subcore")
        wid = cid * NS + sid
        rows_per_tile = NP // NS
        chunks_per_tile = rows_per_tile // B

        pltpu.sync_copy(src_hbm.at[wid], sidx)
        pltpu.sync_copy(dst_hbm.at[wid], didx)

        _zero_rows(g0, B, F)
        @pl.loop(0, chunks_per_tile)
        def _(j):
            pltpu.sync_copy(g0, acc_sh.at[pl.ds(sid * rows_per_tile + j * B, B)])
        plsc.subcore_barrier()

        @pl.loop(0, CH)
        def _(j):
            pltpu.sync_copy(h_hbm.at[sidx.at[j]], g0)
            pltpu.sync_copy(g0, acc_sh.at[didx.at[j]], add=True)

        plsc.subcore_barrier()

        @pl.loop(0, chunks_per_tile)
        def _(j):
            base = sid * rows_per_tile + j * B
            pltpu.sync_copy(acc_sh.at[pl.ds(base, B)], g0)
            pltpu.sync_copy(g0, out_hbm.at[cid].at[pl.ds(base, B)])

    return agg_kernel(h, src_r, dst_r)


def _tc_first(x, W1, degp, grid_n):
    N, F = x.shape

    def body(x_ref, w_ref, d_ref, h_ref, dinv_ref):
        deg = d_ref[0, :, 0] + d_ref[1, :, 0] + 1.0
        dinv = lax.rsqrt(deg)
        h = jnp.dot(x_ref[...], w_ref[...], preferred_element_type=jnp.float32)
        h_ref[...] = h * dinv[:, None]
        dinv_ref[...] = dinv

    return pl.pallas_call(
        body,
        grid=(grid_n,),
        in_specs=[
            pl.BlockSpec((B, F), lambda i: (i, 0)),
            pl.BlockSpec((F, F), lambda i: (0, 0)),
            pl.BlockSpec((NC, B, F), lambda i: (0, i, 0)),
        ],
        out_specs=[
            pl.BlockSpec((B, F), lambda i: (i, 0)),
            pl.BlockSpec((B,), lambda i: (i,)),
        ],
        out_shape=[
            jax.ShapeDtypeStruct((N, F), jnp.float32),
            jax.ShapeDtypeStruct((N,), jnp.float32),
        ],
    )(x, W1, degp)


def _tc_mid(aggp, h1s, dinv, b1, W2, grid_n):
    N, F = h1s.shape

    def body(p_ref, h_ref, dinv_ref, b_ref, w_ref, o_ref):
        agg = p_ref[0] + p_ref[1]
        dinv = dinv_ref[...]
        u = jnp.maximum(dinv[:, None] * (agg + h_ref[...]) + b_ref[...], 0.0)
        h2 = jnp.dot(u, w_ref[...], preferred_element_type=jnp.float32)
        o_ref[...] = h2 * dinv[:, None]

    return pl.pallas_call(
        body,
        grid=(grid_n,),
        in_specs=[
            pl.BlockSpec((NC, B, F), lambda i: (0, i, 0)),
            pl.BlockSpec((B, F), lambda i: (i, 0)),
            pl.BlockSpec((B,), lambda i: (i,)),
            pl.BlockSpec((F,), lambda i: (0,)),
            pl.BlockSpec((F, F), lambda i: (0, 0)),
        ],
        out_specs=pl.BlockSpec((B, F), lambda i: (i, 0)),
        out_shape=jax.ShapeDtypeStruct((N, F), jnp.float32),
    )(aggp, h1s, dinv, b1, W2)


def _tc_last(aggp, h2s, dinv, b2, grid_n):
    N, F = h2s.shape

    def body(p_ref, h_ref, dinv_ref, b_ref, o_ref):
        agg = p_ref[0] + p_ref[1]
        dinv = dinv_ref[...]
        o_ref[...] = dinv[:, None] * (agg + h_ref[...]) + b_ref[...]

    return pl.pallas_call(
        body,
        grid=(grid_n,),
        in_specs=[
            pl.BlockSpec((NC, B, F), lambda i: (0, i, 0)),
            pl.BlockSpec((B, F), lambda i: (i, 0)),
            pl.BlockSpec((B,), lambda i: (i,)),
            pl.BlockSpec((F,), lambda i: (0,)),
        ],
        out_specs=pl.BlockSpec((B, F), lambda i: (i, 0)),
        out_shape=jax.ShapeDtypeStruct((N, F), jnp.float32),
    )(aggp, h2s, dinv, b2)


def kernel(x, ei, W1, b1, W2, b2):
    N, F = x.shape
    E = ei.shape[1]

    epw = -(-E // NW)
    CH = -(-epw // B)
    E_pad = NW * CH * B
    NP = -(-(N + 1) // (NS * B)) * (NS * B)
    grid_n = -(-N // B)

    src = jnp.concatenate(
        [ei[0], jnp.zeros((E_pad - E,), ei.dtype)]).reshape(NW, CH, B)
    dst = jnp.concatenate(
        [ei[1], jnp.full((E_pad - E,), N, ei.dtype)]).reshape(NW, CH, B)

    degp = _sc_degree(dst, NP, CH, F)
    h1s, dinv = _tc_first(x, W1, degp, grid_n)
    agg1 = _sc_aggregate(h1s, src, dst, NP, CH, F)
    h2s = _tc_mid(agg1, h1s, dinv, b1, W2, grid_n)
    agg2 = _sc_aggregate(h2s, src, dst, NP, CH, F)
    return _tc_last(agg2, h2s, dinv, b2, grid_n)
